# baseline (speedup 1.0000x reference)
import jax
import jax.numpy as jnp
from jax.experimental import pallas as pl
from jax.experimental.pallas import tpu as pltpu


def _dma_copy(p_in, a_in, f_in, p_out, a_out, f_out, sem_p, sem_a, sem_f):
    cp = pltpu.make_async_copy(p_in, p_out, sem_p)
    ca = pltpu.make_async_copy(a_in, a_out, sem_a)
    cf = pltpu.make_async_copy(f_in, f_out, sem_f)
    cp.start()
    ca.start()
    cf.start()
    cp.wait()
    ca.wait()
    cf.wait()


def kernel(embed_paper, embed_author, embed_field):
    return pl.pallas_call(
        _dma_copy,
        in_specs=[pl.BlockSpec(memory_space=pltpu.MemorySpace.HBM)] * 3,
        out_specs=(pl.BlockSpec(memory_space=pltpu.MemorySpace.HBM),) * 3,
        out_shape=tuple(
            jax.ShapeDtypeStruct(x.shape, x.dtype)
            for x in (embed_paper, embed_author, embed_field)
        ),
        scratch_shapes=[pltpu.SemaphoreType.DMA] * 3,
    )(embed_paper, embed_author, embed_field)


# speedup vs baseline: 48.1853x; 48.1853x over previous
import jax
import jax.numpy as jnp
from jax.experimental import pallas as pl
from jax.experimental.pallas import tpu as pltpu

_N_PAPER, _N_AUTHOR, _N_FIELD = 100000, 50000, 10000
_EMBED = 128
_CHUNK = 25000


def _chunk_list():
    chunks = []
    for t, n in enumerate((_N_PAPER, _N_AUTHOR, _N_FIELD)):
        off = 0
        while off < n:
            rows = min(_CHUNK, n - off)
            chunks.append((t, off, rows))
            off += rows
    return chunks


def _dma_pipeline(p_in, a_in, f_in, p_out, a_out, f_out,
                  buf0, buf1, sin0, sin1, sout0, sout1):
    srcs = (p_in, a_in, f_in)
    dsts = (p_out, a_out, f_out)
    bufs = (buf0, buf1)
    sins = (sin0, sin1)
    souts = (sout0, sout1)
    chunks = _chunk_list()
    n = len(chunks)

    def in_copy(i):
        t, off, rows = chunks[i]
        return pltpu.make_async_copy(
            srcs[t].at[pl.ds(off, rows), :],
            bufs[i % 2].at[pl.ds(0, rows), :],
            sins[i % 2],
        )

    def out_copy(i):
        t, off, rows = chunks[i]
        return pltpu.make_async_copy(
            bufs[i % 2].at[pl.ds(0, rows), :],
            dsts[t].at[pl.ds(off, rows), :],
            souts[i % 2],
        )

    in_copy(0).start()
    for i in range(n):
        if i + 1 < n:
            if i >= 1:
                out_copy(i - 1).wait()
            in_copy(i + 1).start()
        in_copy(i).wait()
        out_copy(i).start()
    if n >= 2:
        out_copy(n - 2).wait()
    out_copy(n - 1).wait()


def kernel(embed_paper, embed_author, embed_field):
    return pl.pallas_call(
        _dma_pipeline,
        in_specs=[pl.BlockSpec(memory_space=pltpu.MemorySpace.HBM)] * 3,
        out_specs=(pl.BlockSpec(memory_space=pltpu.MemorySpace.HBM),) * 3,
        out_shape=tuple(
            jax.ShapeDtypeStruct(x.shape, x.dtype)
            for x in (embed_paper, embed_author, embed_field)
        ),
        scratch_shapes=[
            pltpu.VMEM((_CHUNK, _EMBED), jnp.float32),
            pltpu.VMEM((_CHUNK, _EMBED), jnp.float32),
            pltpu.SemaphoreType.DMA,
            pltpu.SemaphoreType.DMA,
            pltpu.SemaphoreType.DMA,
            pltpu.SemaphoreType.DMA,
        ],
    )(embed_paper, embed_author, embed_field)


# speedup vs baseline: 48.3886x; 1.0042x over previous
import jax
import jax.numpy as jnp
from jax.experimental import pallas as pl
from jax.experimental.pallas import tpu as pltpu

_N_PAPER, _N_AUTHOR, _N_FIELD = 100000, 50000, 10000
_EMBED = 128
_CHUNK = 12500
_SLOTS = 4


def _chunk_list():
    chunks = []
    for t, n in enumerate((_N_PAPER, _N_AUTHOR, _N_FIELD)):
        off = 0
        while off < n:
            rows = min(_CHUNK, n - off)
            chunks.append((t, off, rows))
            off += rows
    return chunks


def _dma_pipeline(p_in, a_in, f_in, p_out, a_out, f_out, *scratch):
    bufs = scratch[:_SLOTS]
    sins = scratch[_SLOTS:2 * _SLOTS]
    souts = scratch[2 * _SLOTS:]
    srcs = (p_in, a_in, f_in)
    dsts = (p_out, a_out, f_out)
    chunks = _chunk_list()
    n = len(chunks)

    def in_copy(i):
        t, off, rows = chunks[i]
        return pltpu.make_async_copy(
            srcs[t].at[pl.ds(off, rows), :],
            bufs[i % _SLOTS].at[pl.ds(0, rows), :],
            sins[i % _SLOTS],
        )

    def out_copy(i):
        t, off, rows = chunks[i]
        return pltpu.make_async_copy(
            bufs[i % _SLOTS].at[pl.ds(0, rows), :],
            dsts[t].at[pl.ds(off, rows), :],
            souts[i % _SLOTS],
        )

    depth = _SLOTS // 2
    for i in range(min(depth, n)):
        in_copy(i).start()
    for i in range(n):
        j = i + depth
        if j < n:
            if j - _SLOTS >= 0:
                out_copy(j - _SLOTS).wait()
            in_copy(j).start()
        in_copy(i).wait()
        out_copy(i).start()
    for i in range(max(0, n - 2 * depth), n):
        out_copy(i).wait()


def kernel(embed_paper, embed_author, embed_field):
    return pl.pallas_call(
        _dma_pipeline,
        in_specs=[pl.BlockSpec(memory_space=pltpu.MemorySpace.HBM)] * 3,
        out_specs=(pl.BlockSpec(memory_space=pltpu.MemorySpace.HBM),) * 3,
        out_shape=tuple(
            jax.ShapeDtypeStruct(x.shape, x.dtype)
            for x in (embed_paper, embed_author, embed_field)
        ),
        scratch_shapes=(
            [pltpu.VMEM((_CHUNK, _EMBED), jnp.float32)] * _SLOTS
            + [pltpu.SemaphoreType.DMA] * (2 * _SLOTS)
        ),
    )(embed_paper, embed_author, embed_field)
